# baseline (speedup 1.0000x reference)
import jax
import jax.numpy as jnp
from jax import lax
from jax.experimental import pallas as pl
from jax.experimental.pallas import tpu as pltpu
from jax.experimental.pallas import tpu_sc as plsc

FEAT = 64
WIDE = 128
SCALE = 8.0
G = 32
NGB = 8
BLK = NGB * G
NC = 2
NS = 16
NW = NC * NS
UNROLL = 8


def _emb_body(table_hbm, idx_hbm, out_hbm,
              idx_v0, idx_v1, rows_v0, rows_v1, cmp_v,
              isem0, isem1, gsem0, gsem1, osem):
    idx_v = (idx_v0, idx_v1)
    rows_v = (rows_v0, rows_v1)
    isem = (isem0, isem1)
    gsem = (gsem0, gsem1)

    wid = lax.axis_index("s") * NC + lax.axis_index("c")
    blocks = out_hbm.shape[0] // (BLK * NW)
    be0 = wid * blocks

    def load_idx(b, g_blk):
        pltpu.async_copy(
            idx_hbm.at[pl.ds(g_blk * NGB, NGB)], idx_v[b], isem[b]
        )

    def fire(b, g_blk):
        pltpu.make_async_copy(
            idx_hbm.at[pl.ds(g_blk * NGB, NGB)], idx_v[b], isem[b]
        ).wait()
        for j in range(NGB):
            pltpu.async_copy(
                table_hbm.at[idx_v[b].at[j]],
                rows_v[b].at[pl.ds(j * G, G)],
                gsem[b],
            )

    def drain_gathers(b):
        for j in range(NGB):
            pltpu.make_async_copy(
                table_hbm.at[idx_v[b].at[j]],
                rows_v[b].at[pl.ds(j * G, G)],
                gsem[b],
            ).wait()

    def store(g_blk):
        pltpu.async_copy(cmp_v, out_hbm.at[pl.ds(g_blk * BLK, BLK)], osem)

    def wait_store(g_blk):
        pltpu.make_async_copy(
            cmp_v, out_hbm.at[pl.ds(g_blk * BLK, BLK)], osem
        ).wait()

    def scale(b):
        def body(i, c):
            r0 = i * UNROLL
            for dr in range(UNROLL):
                for col in range(FEAT // 16):
                    sl = pl.ds(col * 16, 16)
                    cmp_v[r0 + dr, sl] = rows_v[b][r0 + dr, sl] * SCALE
            return c
        lax.fori_loop(0, BLK // UNROLL, body, 0)

    load_idx(0, be0)
    load_idx(1, be0 + 1)
    fire(0, be0)

    def pair(gp, carry):
        for b in range(2):
            g = gp * 2 + b
            blk = be0 + g
            gn = lax.rem(g + 1, blocks)
            fire(1 - b, be0 + gn)
            drain_gathers(b)
            gn2 = lax.rem(g + 2, blocks)
            load_idx(b, be0 + gn2)
            if b == 0:
                @pl.when(gp > 0)
                def _():
                    wait_store(blk - 1)
            else:
                wait_store(blk - 1)
            scale(b)
            store(blk)
        return carry

    lax.fori_loop(0, blocks // 2, pair, 0)

    pltpu.make_async_copy(
        idx_hbm.at[pl.ds((be0 + 1) * NGB, NGB)], idx_v[1], isem[1]
    ).wait()
    drain_gathers(0)
    wait_store(be0 + blocks - 1)


def kernel(token_idx, table):
    batch, hist = token_idx.shape
    n = batch * hist
    table_p = jnp.pad(table, ((0, 0), (0, WIDE - FEAT)))
    idx = token_idx.reshape(n // G, G).astype(jnp.int32)
    mesh = plsc.VectorSubcoreMesh(core_axis_name="c", subcore_axis_name="s")
    out = pl.kernel(
        _emb_body,
        out_type=jax.ShapeDtypeStruct((n, FEAT), jnp.float32),
        mesh=mesh,
        scratch_types=[
            pltpu.VMEM((NGB, G), jnp.int32),
            pltpu.VMEM((NGB, G), jnp.int32),
            pltpu.VMEM((BLK, WIDE), jnp.float32),
            pltpu.VMEM((BLK, WIDE), jnp.float32),
            pltpu.VMEM((BLK, FEAT), jnp.float32),
            pltpu.SemaphoreType.DMA,
            pltpu.SemaphoreType.DMA,
            pltpu.SemaphoreType.DMA,
            pltpu.SemaphoreType.DMA,
            pltpu.SemaphoreType.DMA,
        ],
        compiler_params=pltpu.CompilerParams(use_tc_tiling_on_sc=True),
    )(table_p, idx)
    return out.reshape(batch, hist, FEAT)
